# baseline (speedup 1.0000x reference)
import functools
import math

import jax
import jax.numpy as jnp
from jax import lax
from jax.experimental import pallas as pl
from jax.experimental.pallas import tpu as pltpu
from jax.experimental.pallas import tpu_sc as plsc

B_ = 16384
A_ = 26
NB_ = 20
D_ = 32
OUT_ = 128
H_ = (A_ * D_) // 2
C_ = A_ * NB_
AD_ = A_ * D_

BT = 512
NT = B_ // BT

AP_ = 32
L_ = 16
NW_ = 32
SLAB_ = B_ // 16
ROWS_ = B_ // NW_

_INV_SQRT2 = 1.0 / math.sqrt(2.0)


def _gelu(x):
    return 0.5 * x * (1.0 + jax.lax.erf(x * _INV_SQRT2))


def _sc_front_body(act_hbm, tb_hbm, out_hbm, act_v, out_v, part_v, allp_v,
                   tb_v, shared):
    c = lax.axis_index("c")
    s = lax.axis_index("s")
    wid = s * 2 + c

    pltpu.sync_copy(act_hbm.at[pl.ds(s * (SLAB_ * AP_), SLAB_ * AP_)], act_v)
    pltpu.sync_copy(tb_hbm, tb_v)

    def p1(i, carry):
        mn0, mn1, mx0, mx1 = carry
        a0 = act_v[pl.ds(i * AP_, L_)]
        a1 = act_v[pl.ds(i * AP_ + L_, L_)]
        return (jnp.minimum(mn0, a0), jnp.minimum(mn1, a1),
                jnp.maximum(mx0, a0), jnp.maximum(mx1, a1))

    big = jnp.full((L_,), jnp.inf, jnp.float32)
    mn0, mn1, mx0, mx1 = lax.fori_loop(0, SLAB_, p1, (big, big, -big, -big))
    part_v[pl.ds(0, L_)] = mn0
    part_v[pl.ds(L_, L_)] = mn1
    part_v[pl.ds(2 * L_, L_)] = mx0
    part_v[pl.ds(3 * L_, L_)] = mx1
    pltpu.sync_copy(part_v, shared.at[pl.ds(s * (4 * L_), 4 * L_)])
    plsc.subcore_barrier()
    pltpu.sync_copy(shared, allp_v)
    for j in range(16):
        o = j * 4 * L_
        if j == 0:
            mn0 = allp_v[pl.ds(o, L_)]
            mn1 = allp_v[pl.ds(o + L_, L_)]
            mx0 = allp_v[pl.ds(o + 2 * L_, L_)]
            mx1 = allp_v[pl.ds(o + 3 * L_, L_)]
        else:
            mn0 = jnp.minimum(mn0, allp_v[pl.ds(o, L_)])
            mn1 = jnp.minimum(mn1, allp_v[pl.ds(o + L_, L_)])
            mx0 = jnp.maximum(mx0, allp_v[pl.ds(o + 2 * L_, L_)])
            mx1 = jnp.maximum(mx1, allp_v[pl.ds(o + 3 * L_, L_)])
    df0 = mx0 - mn0
    df1 = mx1 - mn1

    th0 = [mn0 + df0 * tb_v[pl.ds(k * L_, L_)] for k in range(1, NB_ + 1)]
    th1 = [mn1 + df1 * tb_v[pl.ds(k * L_, L_)] for k in range(1, NB_ + 1)]

    base = c * ROWS_

    def p2(r, _):
        o = (base + r) * AP_
        a0 = act_v[pl.ds(o, L_)]
        a1 = act_v[pl.ds(o + L_, L_)]
        c0 = jnp.zeros((L_,), jnp.float32)
        c1 = jnp.zeros((L_,), jnp.float32)
        for k in range(NB_):
            c0 = c0 + jnp.where(th0[k] < a0, 1.0, 0.0)
            c1 = c1 + jnp.where(th1[k] < a1, 1.0, 0.0)
        out_v[pl.ds(r * AP_, L_)] = jnp.minimum(c0, float(NB_ - 1))
        out_v[pl.ds(r * AP_ + L_, L_)] = jnp.minimum(c1, float(NB_ - 1))
        return 0

    lax.fori_loop(0, ROWS_, p2, 0)
    pltpu.sync_copy(out_v, out_hbm.at[pl.ds(wid * (ROWS_ * AP_), ROWS_ * AP_)])


@functools.partial(
    pl.kernel,
    mesh=plsc.VectorSubcoreMesh(core_axis_name="c", subcore_axis_name="s"),
    out_type=jax.ShapeDtypeStruct((B_ * AP_,), jnp.float32),
    scratch_types=[
        pltpu.VMEM((SLAB_ * AP_,), jnp.float32),
        pltpu.VMEM((ROWS_ * AP_,), jnp.float32),
        pltpu.VMEM((4 * L_,), jnp.float32),
        pltpu.VMEM((16 * 4 * L_,), jnp.float32),
        pltpu.VMEM(((NB_ + 4) * L_,), jnp.float32),
        pltpu.VMEM_SHARED((16 * 4 * L_,), jnp.float32),
    ],
)
def _sc_front(act_hbm, tb_hbm, out_hbm, act_v, out_v, part_v, allp_v, tb_v,
              shared):
    _sc_front_body(act_hbm, tb_hbm, out_hbm, act_v, out_v, part_v, allp_v,
                   tb_v, shared)


def _main_body(binv_ref, tab_ref, W1_ref, b1_ref, W2_ref, b2_ref,
               out_ref, E_ref, M_ref):
    t = pl.program_id(0)

    @pl.when(t == 0)
    def _prep():
        er = jax.lax.broadcasted_iota(jnp.int32, (AP_, C_), 0)
        ec = jax.lax.broadcasted_iota(jnp.int32, (AP_, C_), 1)
        E_ref[...] = jnp.where(ec // NB_ == er, 1.0, 0.0)
        dr = jax.lax.broadcasted_iota(jnp.int32, (D_, AD_), 0)
        dc = jax.lax.broadcasted_iota(jnp.int32, (D_, AD_), 1)
        erep = jnp.where(dc % D_ == dr, 1.0, 0.0)
        mr = jax.lax.broadcasted_iota(jnp.int32, (C_, AD_), 0)
        mc = jax.lax.broadcasted_iota(jnp.int32, (C_, AD_), 1)
        mask = jnp.where(mr // NB_ == mc // D_, 1.0, 0.0)
        t520 = jnp.dot(tab_ref[...], erep,
                       preferred_element_type=jnp.float32) * mask
        M_ref[...] = jnp.dot(t520, W1_ref[...],
                             preferred_element_type=jnp.float32)

    binv = binv_ref[...]
    bin_e = jnp.dot(binv, E_ref[...], preferred_element_type=jnp.float32)
    cidx = jax.lax.broadcasted_iota(jnp.int32, (1, C_), 1)
    jmod = (cidx % NB_).astype(jnp.float32)
    onehot = jnp.where(bin_e == jmod, 1.0, 0.0)
    hpre = jnp.dot(onehot, M_ref[...],
                   preferred_element_type=jnp.float32) + b1_ref[...]
    h = _gelu(hpre)
    o = jnp.dot(h, W2_ref[...], preferred_element_type=jnp.float32)
    out_ref[...] = _gelu(o + b2_ref[...])


def kernel(actions, tables, W1, b1, W2, b2):
    tab520 = tables.reshape(C_, D_)
    tlin = jnp.linspace(0.0, 1.0, NB_ + 1, dtype=jnp.float32)
    tb = jnp.broadcast_to(
        jnp.pad(tlin, (0, 3))[:, None],
        (NB_ + 4, L_)).astype(jnp.float32).reshape(-1)
    act_pad = jnp.pad(actions, ((0, 0), (0, AP_ - A_))).reshape(-1)
    b1r = b1.reshape(1, H_)
    b2r = b2.reshape(1, OUT_)

    binv = _sc_front(act_pad, tb).reshape(B_, AP_)

    out = pl.pallas_call(
        _main_body,
        grid=(NT,),
        in_specs=[
            pl.BlockSpec((BT, AP_), lambda t: (t, 0)),
            pl.BlockSpec((C_, D_), lambda t: (0, 0)),
            pl.BlockSpec((AD_, H_), lambda t: (0, 0)),
            pl.BlockSpec((1, H_), lambda t: (0, 0)),
            pl.BlockSpec((H_, OUT_), lambda t: (0, 0)),
            pl.BlockSpec((1, OUT_), lambda t: (0, 0)),
        ],
        out_specs=pl.BlockSpec((BT, OUT_), lambda t: (t, 0)),
        out_shape=jax.ShapeDtypeStruct((B_, OUT_), jnp.float32),
        scratch_shapes=[
            pltpu.VMEM((AP_, C_), jnp.float32),
            pltpu.VMEM((C_, H_), jnp.float32),
        ],
        compiler_params=pltpu.CompilerParams(
            dimension_semantics=("arbitrary",)),
    )(binv, tab520, W1, b1r, W2, b2r)
    return out


# speedup vs baseline: 1.0733x; 1.0733x over previous
import functools
import math

import jax
import jax.numpy as jnp
from jax import lax
from jax.experimental import pallas as pl
from jax.experimental.pallas import tpu as pltpu
from jax.experimental.pallas import tpu_sc as plsc

B_ = 16384
A_ = 26
NB_ = 20
D_ = 32
OUT_ = 128
H_ = (A_ * D_) // 2
C_ = A_ * NB_
AD_ = A_ * D_

BT = 1024
NT = B_ // BT

AP_ = 32
L_ = 16
NW_ = 32
SLAB_ = B_ // 16
ROWS_ = B_ // NW_

_INV_SQRT2 = 1.0 / math.sqrt(2.0)


def _gelu(x):
    return 0.5 * x * (1.0 + jax.lax.erf(x * _INV_SQRT2))


def _sc_front_body(act_hbm, tb_hbm, out_hbm, act_v, out_v, part_v, allp_v,
                   tb_v, shared):
    c = lax.axis_index("c")
    s = lax.axis_index("s")
    wid = s * 2 + c

    pltpu.sync_copy(act_hbm.at[pl.ds(s * (SLAB_ * AP_), SLAB_ * AP_)], act_v)
    pltpu.sync_copy(tb_hbm, tb_v)

    def p1(i, carry):
        mn0, mn1, mx0, mx1 = carry
        a0 = act_v[pl.ds(i * AP_, L_)]
        a1 = act_v[pl.ds(i * AP_ + L_, L_)]
        return (jnp.minimum(mn0, a0), jnp.minimum(mn1, a1),
                jnp.maximum(mx0, a0), jnp.maximum(mx1, a1))

    big = jnp.full((L_,), jnp.inf, jnp.float32)
    mn0, mn1, mx0, mx1 = lax.fori_loop(0, SLAB_, p1, (big, big, -big, -big),
                                       unroll=8)
    part_v[pl.ds(0, L_)] = mn0
    part_v[pl.ds(L_, L_)] = mn1
    part_v[pl.ds(2 * L_, L_)] = mx0
    part_v[pl.ds(3 * L_, L_)] = mx1
    pltpu.sync_copy(part_v, shared.at[pl.ds(s * (4 * L_), 4 * L_)])
    plsc.subcore_barrier()
    pltpu.sync_copy(shared, allp_v)
    for j in range(16):
        o = j * 4 * L_
        if j == 0:
            mn0 = allp_v[pl.ds(o, L_)]
            mn1 = allp_v[pl.ds(o + L_, L_)]
            mx0 = allp_v[pl.ds(o + 2 * L_, L_)]
            mx1 = allp_v[pl.ds(o + 3 * L_, L_)]
        else:
            mn0 = jnp.minimum(mn0, allp_v[pl.ds(o, L_)])
            mn1 = jnp.minimum(mn1, allp_v[pl.ds(o + L_, L_)])
            mx0 = jnp.maximum(mx0, allp_v[pl.ds(o + 2 * L_, L_)])
            mx1 = jnp.maximum(mx1, allp_v[pl.ds(o + 3 * L_, L_)])
    df0 = mx0 - mn0
    df1 = mx1 - mn1

    th0 = [mn0 + df0 * tb_v[pl.ds(k * L_, L_)] for k in range(1, NB_ + 1)]
    th1 = [mn1 + df1 * tb_v[pl.ds(k * L_, L_)] for k in range(1, NB_ + 1)]

    base = c * ROWS_

    def p2(r, _):
        o = (base + r) * AP_
        a0 = act_v[pl.ds(o, L_)]
        a1 = act_v[pl.ds(o + L_, L_)]
        c0 = jnp.zeros((L_,), jnp.float32)
        c1 = jnp.zeros((L_,), jnp.float32)
        for k in range(NB_):
            c0 = c0 + jnp.where(th0[k] < a0, 1.0, 0.0)
            c1 = c1 + jnp.where(th1[k] < a1, 1.0, 0.0)
        out_v[pl.ds(r * AP_, L_)] = jnp.minimum(c0, float(NB_ - 1))
        out_v[pl.ds(r * AP_ + L_, L_)] = jnp.minimum(c1, float(NB_ - 1))
        return 0

    lax.fori_loop(0, ROWS_, p2, 0, unroll=4)
    pltpu.sync_copy(out_v, out_hbm.at[pl.ds(wid * (ROWS_ * AP_), ROWS_ * AP_)])


@functools.partial(
    pl.kernel,
    mesh=plsc.VectorSubcoreMesh(core_axis_name="c", subcore_axis_name="s"),
    out_type=jax.ShapeDtypeStruct((B_ * AP_,), jnp.float32),
    scratch_types=[
        pltpu.VMEM((SLAB_ * AP_,), jnp.float32),
        pltpu.VMEM((ROWS_ * AP_,), jnp.float32),
        pltpu.VMEM((4 * L_,), jnp.float32),
        pltpu.VMEM((16 * 4 * L_,), jnp.float32),
        pltpu.VMEM(((NB_ + 4) * L_,), jnp.float32),
        pltpu.VMEM_SHARED((16 * 4 * L_,), jnp.float32),
    ],
)
def _sc_front(act_hbm, tb_hbm, out_hbm, act_v, out_v, part_v, allp_v, tb_v,
              shared):
    _sc_front_body(act_hbm, tb_hbm, out_hbm, act_v, out_v, part_v, allp_v,
                   tb_v, shared)


def _main_body(binv_ref, tab_ref, W1_ref, b1_ref, W2_ref, b2_ref,
               out_ref, E_ref, M_ref):
    t = pl.program_id(0)

    @pl.when(t == 0)
    def _prep():
        er = jax.lax.broadcasted_iota(jnp.int32, (AP_, C_), 0)
        ec = jax.lax.broadcasted_iota(jnp.int32, (AP_, C_), 1)
        E_ref[...] = jnp.where(ec // NB_ == er, 1.0, 0.0).astype(jnp.bfloat16)
        dr = jax.lax.broadcasted_iota(jnp.int32, (D_, AD_), 0)
        dc = jax.lax.broadcasted_iota(jnp.int32, (D_, AD_), 1)
        erep = jnp.where(dc % D_ == dr, 1.0, 0.0)
        mr = jax.lax.broadcasted_iota(jnp.int32, (C_, AD_), 0)
        mc = jax.lax.broadcasted_iota(jnp.int32, (C_, AD_), 1)
        mask = jnp.where(mr // NB_ == mc // D_, 1.0, 0.0)
        t520 = jnp.dot(tab_ref[...], erep,
                       preferred_element_type=jnp.float32) * mask
        M_ref[...] = jnp.dot(t520, W1_ref[...],
                             preferred_element_type=jnp.float32
                             ).astype(jnp.bfloat16)

    binv = binv_ref[...].astype(jnp.bfloat16)
    bin_e = jnp.dot(binv, E_ref[...], preferred_element_type=jnp.float32)
    cidx = jax.lax.broadcasted_iota(jnp.int32, (1, C_), 1)
    jmod = (cidx % NB_).astype(jnp.float32)
    onehot = jnp.where(bin_e == jmod, 1.0, 0.0).astype(jnp.bfloat16)
    hpre = jnp.dot(onehot, M_ref[...],
                   preferred_element_type=jnp.float32) + b1_ref[...]
    h = _gelu(hpre).astype(jnp.bfloat16)
    o = jnp.dot(h, W2_ref[...], preferred_element_type=jnp.float32)
    out_ref[...] = _gelu(o + b2_ref[...])


def kernel(actions, tables, W1, b1, W2, b2):
    tab520 = tables.reshape(C_, D_)
    tlin = jnp.linspace(0.0, 1.0, NB_ + 1, dtype=jnp.float32)
    tb = jnp.broadcast_to(
        jnp.pad(tlin, (0, 3))[:, None],
        (NB_ + 4, L_)).astype(jnp.float32).reshape(-1)
    act_pad = jnp.pad(actions, ((0, 0), (0, AP_ - A_))).reshape(-1)
    b1r = b1.reshape(1, H_)
    b2r = b2.reshape(1, OUT_)

    binv = _sc_front(act_pad, tb).reshape(B_, AP_)

    out = pl.pallas_call(
        _main_body,
        grid=(NT,),
        in_specs=[
            pl.BlockSpec((BT, AP_), lambda t: (t, 0)),
            pl.BlockSpec((C_, D_), lambda t: (0, 0)),
            pl.BlockSpec((AD_, H_), lambda t: (0, 0)),
            pl.BlockSpec((1, H_), lambda t: (0, 0)),
            pl.BlockSpec((H_, OUT_), lambda t: (0, 0)),
            pl.BlockSpec((1, OUT_), lambda t: (0, 0)),
        ],
        out_specs=pl.BlockSpec((BT, OUT_), lambda t: (t, 0)),
        out_shape=jax.ShapeDtypeStruct((B_, OUT_), jnp.float32),
        scratch_shapes=[
            pltpu.VMEM((AP_, C_), jnp.bfloat16),
            pltpu.VMEM((C_, H_), jnp.bfloat16),
        ],
        compiler_params=pltpu.CompilerParams(
            dimension_semantics=("arbitrary",)),
    )(binv, tab520, W1, b1r, W2, b2r)
    return out


# speedup vs baseline: 1.3755x; 1.2815x over previous
import functools
import math

import jax
import jax.numpy as jnp
from jax import lax
from jax.experimental import pallas as pl
from jax.experimental.pallas import tpu as pltpu
from jax.experimental.pallas import tpu_sc as plsc

B_ = 16384
A_ = 26
NB_ = 20
D_ = 32
OUT_ = 128
H_ = (A_ * D_) // 2
C_ = A_ * NB_
AD_ = A_ * D_

BT = 1024
NT = B_ // BT

AP_ = 32
L_ = 16
NW_ = 32
SLAB_ = B_ // 16
ROWS_ = B_ // NW_

_INV_SQRT2 = 1.0 / math.sqrt(2.0)


def _gelu(x):
    return 0.5 * x * (1.0 + jax.lax.erf(x * _INV_SQRT2))


def _sc_front_body(act_hbm, tb_hbm, out_hbm, act_v, out_v, part_v, allp_v,
                   tb_v, shared):
    c = lax.axis_index("c")
    s = lax.axis_index("s")
    wid = s * 2 + c

    pltpu.sync_copy(act_hbm.at[pl.ds(s * (SLAB_ * AP_), SLAB_ * AP_)], act_v)
    pltpu.sync_copy(tb_hbm, tb_v)

    def p1(i, carry):
        mn0, mn1, mx0, mx1 = carry
        a0 = act_v[pl.ds(i * AP_, L_)]
        a1 = act_v[pl.ds(i * AP_ + L_, L_)]
        return (jnp.minimum(mn0, a0), jnp.minimum(mn1, a1),
                jnp.maximum(mx0, a0), jnp.maximum(mx1, a1))

    big = jnp.full((L_,), jnp.inf, jnp.float32)
    mn0, mn1, mx0, mx1 = lax.fori_loop(0, SLAB_, p1, (big, big, -big, -big),
                                       unroll=8)
    part_v[pl.ds(0, L_)] = mn0
    part_v[pl.ds(L_, L_)] = mn1
    part_v[pl.ds(2 * L_, L_)] = mx0
    part_v[pl.ds(3 * L_, L_)] = mx1
    pltpu.sync_copy(part_v, shared.at[pl.ds(s * (4 * L_), 4 * L_)])
    plsc.subcore_barrier()
    pltpu.sync_copy(shared, allp_v)
    for j in range(16):
        o = j * 4 * L_
        if j == 0:
            mn0 = allp_v[pl.ds(o, L_)]
            mn1 = allp_v[pl.ds(o + L_, L_)]
            mx0 = allp_v[pl.ds(o + 2 * L_, L_)]
            mx1 = allp_v[pl.ds(o + 3 * L_, L_)]
        else:
            mn0 = jnp.minimum(mn0, allp_v[pl.ds(o, L_)])
            mn1 = jnp.minimum(mn1, allp_v[pl.ds(o + L_, L_)])
            mx0 = jnp.maximum(mx0, allp_v[pl.ds(o + 2 * L_, L_)])
            mx1 = jnp.maximum(mx1, allp_v[pl.ds(o + 3 * L_, L_)])
    df0 = mx0 - mn0
    df1 = mx1 - mn1

    th0 = [mn0 + df0 * tb_v[pl.ds(k * L_, L_)] for k in range(1, NB_ + 1)]
    th1 = [mn1 + df1 * tb_v[pl.ds(k * L_, L_)] for k in range(1, NB_ + 1)]

    base = c * ROWS_

    def p2(r, _):
        o = (base + r) * AP_
        a0 = act_v[pl.ds(o, L_)]
        a1 = act_v[pl.ds(o + L_, L_)]
        c0 = jnp.zeros((L_,), jnp.float32)
        c1 = jnp.zeros((L_,), jnp.float32)
        for k in range(NB_):
            c0 = c0 + jnp.where(th0[k] < a0, 1.0, 0.0)
            c1 = c1 + jnp.where(th1[k] < a1, 1.0, 0.0)
        out_v[pl.ds(r * AP_, L_)] = jnp.minimum(c0, float(NB_ - 1))
        out_v[pl.ds(r * AP_ + L_, L_)] = jnp.minimum(c1, float(NB_ - 1))
        return 0

    lax.fori_loop(0, ROWS_, p2, 0, unroll=4)
    pltpu.sync_copy(out_v, out_hbm.at[pl.ds(wid * (ROWS_ * AP_), ROWS_ * AP_)])


@functools.partial(
    pl.kernel,
    mesh=plsc.VectorSubcoreMesh(core_axis_name="c", subcore_axis_name="s"),
    out_type=jax.ShapeDtypeStruct((B_ * AP_,), jnp.float32),
    scratch_types=[
        pltpu.VMEM((SLAB_ * AP_,), jnp.float32),
        pltpu.VMEM((ROWS_ * AP_,), jnp.float32),
        pltpu.VMEM((4 * L_,), jnp.float32),
        pltpu.VMEM((16 * 4 * L_,), jnp.float32),
        pltpu.VMEM(((NB_ + 4) * L_,), jnp.float32),
        pltpu.VMEM_SHARED((16 * 4 * L_,), jnp.float32),
    ],
)
def _sc_front(act_hbm, tb_hbm, out_hbm, act_v, out_v, part_v, allp_v, tb_v,
              shared):
    _sc_front_body(act_hbm, tb_hbm, out_hbm, act_v, out_v, part_v, allp_v,
                   tb_v, shared)


def _minmax_body(act_ref, mm_ref):
    t = pl.program_id(0)
    act = act_ref[...]
    mn = jnp.min(act, axis=0, keepdims=True)
    mx = jnp.max(act, axis=0, keepdims=True)
    cur = jnp.concatenate([mn, -mx], axis=0)

    @pl.when(t == 0)
    def _init():
        mm_ref[...] = cur

    @pl.when(t != 0)
    def _acc():
        mm_ref[...] = jnp.minimum(mm_ref[...], cur)


def _main_body(tlin_ref, mm_ref, act_ref, tab_ref, W1_ref, b1_ref, W2_ref,
               b2_ref, out_ref, E_ref, M_ref):
    t = pl.program_id(0)

    @pl.when(t == 0)
    def _prep():
        er = jax.lax.broadcasted_iota(jnp.int32, (AP_, C_), 0)
        ec = jax.lax.broadcasted_iota(jnp.int32, (AP_, C_), 1)
        E_ref[...] = jnp.where(ec // NB_ == er, 1.0, 0.0).astype(jnp.bfloat16)
        dr = jax.lax.broadcasted_iota(jnp.int32, (D_, AD_), 0)
        dc = jax.lax.broadcasted_iota(jnp.int32, (D_, AD_), 1)
        erep = jnp.where(dc % D_ == dr, 1.0, 0.0)
        mr = jax.lax.broadcasted_iota(jnp.int32, (C_, AD_), 0)
        mc = jax.lax.broadcasted_iota(jnp.int32, (C_, AD_), 1)
        mask = jnp.where(mr // NB_ == mc // D_, 1.0, 0.0)
        t520 = jnp.dot(tab_ref[...], erep,
                       preferred_element_type=jnp.float32) * mask
        M_ref[...] = jnp.dot(t520, W1_ref[...],
                             preferred_element_type=jnp.float32
                             ).astype(jnp.bfloat16)

    act = act_ref[...]
    mn = mm_ref[0:1, :]
    diff = (-mm_ref[1:2, :]) - mn
    cnt = jnp.zeros_like(act)
    for k in range(1, NB_ + 1):
        th = mn + diff * tlin_ref[0, k]
        cnt = cnt + jnp.where(th < act, 1.0, 0.0)
    binv = jnp.minimum(cnt, float(NB_ - 1)).astype(jnp.bfloat16)
    bin_e = jnp.dot(binv, E_ref[...], preferred_element_type=jnp.float32)
    cidx = jax.lax.broadcasted_iota(jnp.int32, (1, C_), 1)
    jmod = (cidx % NB_).astype(jnp.float32)
    onehot = jnp.where(bin_e == jmod, 1.0, 0.0).astype(jnp.bfloat16)
    hpre = jnp.dot(onehot, M_ref[...],
                   preferred_element_type=jnp.float32) + b1_ref[...]
    h = _gelu(hpre).astype(jnp.bfloat16)
    o = jnp.dot(h, W2_ref[...], preferred_element_type=jnp.float32)
    out_ref[...] = _gelu(o + b2_ref[...])


def kernel(actions, tables, W1, b1, W2, b2):
    tab520 = tables.reshape(C_, D_)
    tlin = jnp.linspace(0.0, 1.0, NB_ + 1, dtype=jnp.float32)
    tb = jnp.broadcast_to(
        jnp.pad(tlin, (0, 3))[:, None],
        (NB_ + 4, L_)).astype(jnp.float32).reshape(-1)
    act_pad = jnp.pad(actions, ((0, 0), (0, AP_ - A_))).reshape(-1)
    b1r = b1.reshape(1, H_)
    b2r = b2.reshape(1, OUT_)

    act2d = act_pad.reshape(B_, AP_)
    mm = pl.pallas_call(
        _minmax_body,
        grid=(NT,),
        in_specs=[pl.BlockSpec((BT, AP_), lambda t: (t, 0))],
        out_specs=pl.BlockSpec((2, AP_), lambda t: (0, 0)),
        out_shape=jax.ShapeDtypeStruct((2, AP_), jnp.float32),
        compiler_params=pltpu.CompilerParams(
            dimension_semantics=("arbitrary",)),
    )(act2d)

    tlin2 = tlin.reshape(1, NB_ + 1)
    out = pl.pallas_call(
        _main_body,
        grid=(NT,),
        in_specs=[
            pl.BlockSpec((1, NB_ + 1), lambda t: (0, 0)),
            pl.BlockSpec((2, AP_), lambda t: (0, 0)),
            pl.BlockSpec((BT, AP_), lambda t: (t, 0)),
            pl.BlockSpec((C_, D_), lambda t: (0, 0)),
            pl.BlockSpec((AD_, H_), lambda t: (0, 0)),
            pl.BlockSpec((1, H_), lambda t: (0, 0)),
            pl.BlockSpec((H_, OUT_), lambda t: (0, 0)),
            pl.BlockSpec((1, OUT_), lambda t: (0, 0)),
        ],
        out_specs=pl.BlockSpec((BT, OUT_), lambda t: (t, 0)),
        out_shape=jax.ShapeDtypeStruct((B_, OUT_), jnp.float32),
        scratch_shapes=[
            pltpu.VMEM((AP_, C_), jnp.bfloat16),
            pltpu.VMEM((C_, H_), jnp.bfloat16),
        ],
        compiler_params=pltpu.CompilerParams(
            dimension_semantics=("arbitrary",)),
    )(tlin2, mm, act2d, tab520, W1, b1r, W2, b2r)
    return out


# speedup vs baseline: 1.6779x; 1.2199x over previous
import functools
import math

import jax
import jax.numpy as jnp
from jax import lax
from jax.experimental import pallas as pl
from jax.experimental.pallas import tpu as pltpu
from jax.experimental.pallas import tpu_sc as plsc

B_ = 16384
A_ = 26
NB_ = 20
D_ = 32
OUT_ = 128
H_ = (A_ * D_) // 2
C_ = A_ * NB_
AD_ = A_ * D_

BT = 2048
NT = B_ // BT

AP_ = 32
L_ = 16
NW_ = 32
SLAB_ = B_ // 16
ROWS_ = B_ // NW_

_INV_SQRT2 = 1.0 / math.sqrt(2.0)


def _gelu(x):
    return 0.5 * x * (1.0 + jax.lax.erf(x * _INV_SQRT2))


def _sc_front_body(act_hbm, tb_hbm, out_hbm, act_v, out_v, part_v, allp_v,
                   tb_v, shared):
    c = lax.axis_index("c")
    s = lax.axis_index("s")
    wid = s * 2 + c

    pltpu.sync_copy(act_hbm.at[pl.ds(s * (SLAB_ * AP_), SLAB_ * AP_)], act_v)
    pltpu.sync_copy(tb_hbm, tb_v)

    def p1(i, carry):
        mn0, mn1, mx0, mx1 = carry
        a0 = act_v[pl.ds(i * AP_, L_)]
        a1 = act_v[pl.ds(i * AP_ + L_, L_)]
        return (jnp.minimum(mn0, a0), jnp.minimum(mn1, a1),
                jnp.maximum(mx0, a0), jnp.maximum(mx1, a1))

    big = jnp.full((L_,), jnp.inf, jnp.float32)
    mn0, mn1, mx0, mx1 = lax.fori_loop(0, SLAB_, p1, (big, big, -big, -big),
                                       unroll=8)
    part_v[pl.ds(0, L_)] = mn0
    part_v[pl.ds(L_, L_)] = mn1
    part_v[pl.ds(2 * L_, L_)] = mx0
    part_v[pl.ds(3 * L_, L_)] = mx1
    pltpu.sync_copy(part_v, shared.at[pl.ds(s * (4 * L_), 4 * L_)])
    plsc.subcore_barrier()
    pltpu.sync_copy(shared, allp_v)
    for j in range(16):
        o = j * 4 * L_
        if j == 0:
            mn0 = allp_v[pl.ds(o, L_)]
            mn1 = allp_v[pl.ds(o + L_, L_)]
            mx0 = allp_v[pl.ds(o + 2 * L_, L_)]
            mx1 = allp_v[pl.ds(o + 3 * L_, L_)]
        else:
            mn0 = jnp.minimum(mn0, allp_v[pl.ds(o, L_)])
            mn1 = jnp.minimum(mn1, allp_v[pl.ds(o + L_, L_)])
            mx0 = jnp.maximum(mx0, allp_v[pl.ds(o + 2 * L_, L_)])
            mx1 = jnp.maximum(mx1, allp_v[pl.ds(o + 3 * L_, L_)])
    df0 = mx0 - mn0
    df1 = mx1 - mn1

    th0 = [mn0 + df0 * tb_v[pl.ds(k * L_, L_)] for k in range(1, NB_ + 1)]
    th1 = [mn1 + df1 * tb_v[pl.ds(k * L_, L_)] for k in range(1, NB_ + 1)]

    base = c * ROWS_

    def p2(r, _):
        o = (base + r) * AP_
        a0 = act_v[pl.ds(o, L_)]
        a1 = act_v[pl.ds(o + L_, L_)]
        c0 = jnp.zeros((L_,), jnp.float32)
        c1 = jnp.zeros((L_,), jnp.float32)
        for k in range(NB_):
            c0 = c0 + jnp.where(th0[k] < a0, 1.0, 0.0)
            c1 = c1 + jnp.where(th1[k] < a1, 1.0, 0.0)
        out_v[pl.ds(r * AP_, L_)] = jnp.minimum(c0, float(NB_ - 1))
        out_v[pl.ds(r * AP_ + L_, L_)] = jnp.minimum(c1, float(NB_ - 1))
        return 0

    lax.fori_loop(0, ROWS_, p2, 0, unroll=4)
    pltpu.sync_copy(out_v, out_hbm.at[pl.ds(wid * (ROWS_ * AP_), ROWS_ * AP_)])


@functools.partial(
    pl.kernel,
    mesh=plsc.VectorSubcoreMesh(core_axis_name="c", subcore_axis_name="s"),
    out_type=jax.ShapeDtypeStruct((B_ * AP_,), jnp.float32),
    scratch_types=[
        pltpu.VMEM((SLAB_ * AP_,), jnp.float32),
        pltpu.VMEM((ROWS_ * AP_,), jnp.float32),
        pltpu.VMEM((4 * L_,), jnp.float32),
        pltpu.VMEM((16 * 4 * L_,), jnp.float32),
        pltpu.VMEM(((NB_ + 4) * L_,), jnp.float32),
        pltpu.VMEM_SHARED((16 * 4 * L_,), jnp.float32),
    ],
)
def _sc_front(act_hbm, tb_hbm, out_hbm, act_v, out_v, part_v, allp_v, tb_v,
              shared):
    _sc_front_body(act_hbm, tb_hbm, out_hbm, act_v, out_v, part_v, allp_v,
                   tb_v, shared)


def _minmax_body(act_ref, mm_ref):
    t = pl.program_id(0)
    act = act_ref[...]
    mn = jnp.min(act, axis=0, keepdims=True)
    mx = jnp.max(act, axis=0, keepdims=True)
    cur = jnp.concatenate([mn, -mx], axis=0)

    @pl.when(t == 0)
    def _init():
        mm_ref[...] = cur

    @pl.when(t != 0)
    def _acc():
        mm_ref[...] = jnp.minimum(mm_ref[...], cur)


def _main_body(tlin_ref, mm_ref, act_ref, tab_ref, W1_ref, b1_ref, W2_ref,
               b2_ref, out_ref, E_ref, M_ref):
    t = pl.program_id(0)

    @pl.when(t == 0)
    def _prep():
        er = jax.lax.broadcasted_iota(jnp.int32, (A_, C_), 0)
        ec = jax.lax.broadcasted_iota(jnp.int32, (A_, C_), 1)
        E_ref[...] = jnp.where(ec // NB_ == er, 1.0, 0.0).astype(jnp.bfloat16)
        dr = jax.lax.broadcasted_iota(jnp.int32, (D_, AD_), 0)
        dc = jax.lax.broadcasted_iota(jnp.int32, (D_, AD_), 1)
        erep = jnp.where(dc % D_ == dr, 1.0, 0.0)
        mr = jax.lax.broadcasted_iota(jnp.int32, (C_, AD_), 0)
        mc = jax.lax.broadcasted_iota(jnp.int32, (C_, AD_), 1)
        mask = jnp.where(mr // NB_ == mc // D_, 1.0, 0.0)
        t520 = jnp.dot(tab_ref[...], erep,
                       preferred_element_type=jnp.float32) * mask
        M_ref[...] = jnp.dot(t520, W1_ref[...],
                             preferred_element_type=jnp.float32
                             ).astype(jnp.bfloat16)

    act = act_ref[...]
    mn = mm_ref[0:1, :]
    diff = (-mm_ref[1:2, :]) - mn
    cnt = jnp.zeros_like(act)
    for k in range(1, NB_ + 1):
        th = mn + diff * tlin_ref[0, k]
        cnt = cnt + jnp.where(th < act, 1.0, 0.0)
    binv = jnp.minimum(cnt, float(NB_ - 1)).astype(jnp.bfloat16)
    bin_e = jnp.dot(binv, E_ref[...], preferred_element_type=jnp.float32)
    cidx = jax.lax.broadcasted_iota(jnp.int32, (1, C_), 1)
    jmod = (cidx % NB_).astype(jnp.float32)
    onehot = jnp.where(bin_e == jmod, 1.0, 0.0).astype(jnp.bfloat16)
    hpre = jnp.dot(onehot, M_ref[...],
                   preferred_element_type=jnp.float32) + b1_ref[...]
    h = _gelu(hpre).astype(jnp.bfloat16)
    o = jnp.dot(h, W2_ref[...], preferred_element_type=jnp.float32)
    out_ref[...] = _gelu(o + b2_ref[...])


def kernel(actions, tables, W1, b1, W2, b2):
    tab520 = tables.reshape(C_, D_)
    tlin = jnp.linspace(0.0, 1.0, NB_ + 1, dtype=jnp.float32)
    tb = jnp.broadcast_to(
        jnp.pad(tlin, (0, 3))[:, None],
        (NB_ + 4, L_)).astype(jnp.float32).reshape(-1)
    act_pad = jnp.pad(actions, ((0, 0), (0, AP_ - A_))).reshape(-1)
    b1r = b1.reshape(1, H_)
    b2r = b2.reshape(1, OUT_)

    mm = pl.pallas_call(
        _minmax_body,
        grid=(NT,),
        in_specs=[pl.BlockSpec((BT, A_), lambda t: (t, 0))],
        out_specs=pl.BlockSpec((2, A_), lambda t: (0, 0)),
        out_shape=jax.ShapeDtypeStruct((2, A_), jnp.float32),
        compiler_params=pltpu.CompilerParams(
            dimension_semantics=("arbitrary",)),
    )(actions)

    tlin2 = tlin.reshape(1, NB_ + 1)
    out = pl.pallas_call(
        _main_body,
        grid=(NT,),
        in_specs=[
            pl.BlockSpec((1, NB_ + 1), lambda t: (0, 0)),
            pl.BlockSpec((2, A_), lambda t: (0, 0)),
            pl.BlockSpec((BT, A_), lambda t: (t, 0)),
            pl.BlockSpec((C_, D_), lambda t: (0, 0)),
            pl.BlockSpec((AD_, H_), lambda t: (0, 0)),
            pl.BlockSpec((1, H_), lambda t: (0, 0)),
            pl.BlockSpec((H_, OUT_), lambda t: (0, 0)),
            pl.BlockSpec((1, OUT_), lambda t: (0, 0)),
        ],
        out_specs=pl.BlockSpec((BT, OUT_), lambda t: (t, 0)),
        out_shape=jax.ShapeDtypeStruct((B_, OUT_), jnp.float32),
        scratch_shapes=[
            pltpu.VMEM((A_, C_), jnp.bfloat16),
            pltpu.VMEM((C_, H_), jnp.bfloat16),
        ],
        compiler_params=pltpu.CompilerParams(
            dimension_semantics=("arbitrary",)),
    )(tlin2, mm, actions, tab520, W1, b1r, W2, b2r)
    return out


# speedup vs baseline: 1.7588x; 1.0482x over previous
import functools
import math

import jax
import jax.numpy as jnp
from jax import lax
from jax.experimental import pallas as pl
from jax.experimental.pallas import tpu as pltpu
from jax.experimental.pallas import tpu_sc as plsc

B_ = 16384
A_ = 26
NB_ = 20
D_ = 32
OUT_ = 128
H_ = (A_ * D_) // 2
C_ = A_ * NB_
AD_ = A_ * D_

BT = 4096
NT = B_ // BT

AP_ = 32
L_ = 16
NW_ = 32
SLAB_ = B_ // 16
ROWS_ = B_ // NW_

_INV_SQRT2 = 1.0 / math.sqrt(2.0)


def _gelu(x):
    return 0.5 * x * (1.0 + jax.lax.erf(x * _INV_SQRT2))


def _sc_front_body(act_hbm, tb_hbm, out_hbm, act_v, out_v, part_v, allp_v,
                   tb_v, shared):
    c = lax.axis_index("c")
    s = lax.axis_index("s")
    wid = s * 2 + c

    pltpu.sync_copy(act_hbm.at[pl.ds(s * (SLAB_ * AP_), SLAB_ * AP_)], act_v)
    pltpu.sync_copy(tb_hbm, tb_v)

    def p1(i, carry):
        mn0, mn1, mx0, mx1 = carry
        a0 = act_v[pl.ds(i * AP_, L_)]
        a1 = act_v[pl.ds(i * AP_ + L_, L_)]
        return (jnp.minimum(mn0, a0), jnp.minimum(mn1, a1),
                jnp.maximum(mx0, a0), jnp.maximum(mx1, a1))

    big = jnp.full((L_,), jnp.inf, jnp.float32)
    mn0, mn1, mx0, mx1 = lax.fori_loop(0, SLAB_, p1, (big, big, -big, -big),
                                       unroll=8)
    part_v[pl.ds(0, L_)] = mn0
    part_v[pl.ds(L_, L_)] = mn1
    part_v[pl.ds(2 * L_, L_)] = mx0
    part_v[pl.ds(3 * L_, L_)] = mx1
    pltpu.sync_copy(part_v, shared.at[pl.ds(s * (4 * L_), 4 * L_)])
    plsc.subcore_barrier()
    pltpu.sync_copy(shared, allp_v)
    for j in range(16):
        o = j * 4 * L_
        if j == 0:
            mn0 = allp_v[pl.ds(o, L_)]
            mn1 = allp_v[pl.ds(o + L_, L_)]
            mx0 = allp_v[pl.ds(o + 2 * L_, L_)]
            mx1 = allp_v[pl.ds(o + 3 * L_, L_)]
        else:
            mn0 = jnp.minimum(mn0, allp_v[pl.ds(o, L_)])
            mn1 = jnp.minimum(mn1, allp_v[pl.ds(o + L_, L_)])
            mx0 = jnp.maximum(mx0, allp_v[pl.ds(o + 2 * L_, L_)])
            mx1 = jnp.maximum(mx1, allp_v[pl.ds(o + 3 * L_, L_)])
    df0 = mx0 - mn0
    df1 = mx1 - mn1

    th0 = [mn0 + df0 * tb_v[pl.ds(k * L_, L_)] for k in range(1, NB_ + 1)]
    th1 = [mn1 + df1 * tb_v[pl.ds(k * L_, L_)] for k in range(1, NB_ + 1)]

    base = c * ROWS_

    def p2(r, _):
        o = (base + r) * AP_
        a0 = act_v[pl.ds(o, L_)]
        a1 = act_v[pl.ds(o + L_, L_)]
        c0 = jnp.zeros((L_,), jnp.float32)
        c1 = jnp.zeros((L_,), jnp.float32)
        for k in range(NB_):
            c0 = c0 + jnp.where(th0[k] < a0, 1.0, 0.0)
            c1 = c1 + jnp.where(th1[k] < a1, 1.0, 0.0)
        out_v[pl.ds(r * AP_, L_)] = jnp.minimum(c0, float(NB_ - 1))
        out_v[pl.ds(r * AP_ + L_, L_)] = jnp.minimum(c1, float(NB_ - 1))
        return 0

    lax.fori_loop(0, ROWS_, p2, 0, unroll=4)
    pltpu.sync_copy(out_v, out_hbm.at[pl.ds(wid * (ROWS_ * AP_), ROWS_ * AP_)])


@functools.partial(
    pl.kernel,
    mesh=plsc.VectorSubcoreMesh(core_axis_name="c", subcore_axis_name="s"),
    out_type=jax.ShapeDtypeStruct((B_ * AP_,), jnp.float32),
    scratch_types=[
        pltpu.VMEM((SLAB_ * AP_,), jnp.float32),
        pltpu.VMEM((ROWS_ * AP_,), jnp.float32),
        pltpu.VMEM((4 * L_,), jnp.float32),
        pltpu.VMEM((16 * 4 * L_,), jnp.float32),
        pltpu.VMEM(((NB_ + 4) * L_,), jnp.float32),
        pltpu.VMEM_SHARED((16 * 4 * L_,), jnp.float32),
    ],
)
def _sc_front(act_hbm, tb_hbm, out_hbm, act_v, out_v, part_v, allp_v, tb_v,
              shared):
    _sc_front_body(act_hbm, tb_hbm, out_hbm, act_v, out_v, part_v, allp_v,
                   tb_v, shared)


def _minmax_body(act_ref, mm_ref):
    t = pl.program_id(0)
    act = act_ref[...]
    mn = jnp.min(act, axis=0, keepdims=True)
    mx = jnp.max(act, axis=0, keepdims=True)
    cur = jnp.concatenate([mn, -mx], axis=0)

    @pl.when(t == 0)
    def _init():
        mm_ref[...] = cur

    @pl.when(t != 0)
    def _acc():
        mm_ref[...] = jnp.minimum(mm_ref[...], cur)


def _main_body(tlin_ref, mm_ref, act_ref, tab_ref, W1_ref, b1_ref, W2_ref,
               b2_ref, out_ref, E_ref, M_ref):
    t = pl.program_id(0)

    @pl.when(t == 0)
    def _prep():
        er = jax.lax.broadcasted_iota(jnp.int32, (A_, C_), 0)
        ec = jax.lax.broadcasted_iota(jnp.int32, (A_, C_), 1)
        E_ref[...] = jnp.where(ec // NB_ == er, 1.0, 0.0).astype(jnp.bfloat16)
        dr = jax.lax.broadcasted_iota(jnp.int32, (D_, AD_), 0)
        dc = jax.lax.broadcasted_iota(jnp.int32, (D_, AD_), 1)
        erep = jnp.where(dc % D_ == dr, 1.0, 0.0)
        mr = jax.lax.broadcasted_iota(jnp.int32, (C_, AD_), 0)
        mc = jax.lax.broadcasted_iota(jnp.int32, (C_, AD_), 1)
        mask = jnp.where(mr // NB_ == mc // D_, 1.0, 0.0)
        t520 = jnp.dot(tab_ref[...], erep,
                       preferred_element_type=jnp.float32) * mask
        M_ref[...] = jnp.dot(t520, W1_ref[...],
                             preferred_element_type=jnp.float32
                             ).astype(jnp.bfloat16)

    act = act_ref[...]
    mn = mm_ref[0:1, :]
    diff = (-mm_ref[1:2, :]) - mn
    cnt = jnp.zeros_like(act)
    for k in range(1, NB_ + 1):
        th = mn + diff * tlin_ref[0, k]
        cnt = cnt + jnp.where(th < act, 1.0, 0.0)
    binv = jnp.minimum(cnt, float(NB_ - 1)).astype(jnp.bfloat16)
    bin_e = jnp.dot(binv, E_ref[...], preferred_element_type=jnp.float32)
    cidx = jax.lax.broadcasted_iota(jnp.int32, (1, C_), 1)
    jmod = (cidx % NB_).astype(jnp.float32)
    onehot = jnp.where(bin_e == jmod, 1.0, 0.0).astype(jnp.bfloat16)
    hpre = jnp.dot(onehot, M_ref[...],
                   preferred_element_type=jnp.float32) + b1_ref[...]
    h = _gelu(hpre).astype(jnp.bfloat16)
    o = jnp.dot(h, W2_ref[...], preferred_element_type=jnp.float32)
    out_ref[...] = _gelu(o + b2_ref[...])


def kernel(actions, tables, W1, b1, W2, b2):
    tab520 = tables.reshape(C_, D_)
    tlin = jnp.linspace(0.0, 1.0, NB_ + 1, dtype=jnp.float32)
    tb = jnp.broadcast_to(
        jnp.pad(tlin, (0, 3))[:, None],
        (NB_ + 4, L_)).astype(jnp.float32).reshape(-1)
    act_pad = jnp.pad(actions, ((0, 0), (0, AP_ - A_))).reshape(-1)
    b1r = b1.reshape(1, H_)
    b2r = b2.reshape(1, OUT_)

    mm = pl.pallas_call(
        _minmax_body,
        grid=(NT,),
        in_specs=[pl.BlockSpec((BT, A_), lambda t: (t, 0))],
        out_specs=pl.BlockSpec((2, A_), lambda t: (0, 0)),
        out_shape=jax.ShapeDtypeStruct((2, A_), jnp.float32),
        compiler_params=pltpu.CompilerParams(
            dimension_semantics=("arbitrary",)),
    )(actions)

    tlin2 = tlin.reshape(1, NB_ + 1)
    out = pl.pallas_call(
        _main_body,
        grid=(NT,),
        in_specs=[
            pl.BlockSpec((1, NB_ + 1), lambda t: (0, 0)),
            pl.BlockSpec((2, A_), lambda t: (0, 0)),
            pl.BlockSpec((BT, A_), lambda t: (t, 0)),
            pl.BlockSpec((C_, D_), lambda t: (0, 0)),
            pl.BlockSpec((AD_, H_), lambda t: (0, 0)),
            pl.BlockSpec((1, H_), lambda t: (0, 0)),
            pl.BlockSpec((H_, OUT_), lambda t: (0, 0)),
            pl.BlockSpec((1, OUT_), lambda t: (0, 0)),
        ],
        out_specs=pl.BlockSpec((BT, OUT_), lambda t: (t, 0)),
        out_shape=jax.ShapeDtypeStruct((B_, OUT_), jnp.float32),
        scratch_shapes=[
            pltpu.VMEM((A_, C_), jnp.bfloat16),
            pltpu.VMEM((C_, H_), jnp.bfloat16),
        ],
        compiler_params=pltpu.CompilerParams(
            dimension_semantics=("arbitrary",)),
    )(tlin2, mm, actions, tab520, W1, b1r, W2, b2r)
    return out


# speedup vs baseline: 1.7605x; 1.0010x over previous
import functools
import math

import jax
import jax.numpy as jnp
from jax import lax
from jax.experimental import pallas as pl
from jax.experimental.pallas import tpu as pltpu
from jax.experimental.pallas import tpu_sc as plsc

B_ = 16384
A_ = 26
NB_ = 20
D_ = 32
OUT_ = 128
H_ = (A_ * D_) // 2
C_ = A_ * NB_
AD_ = A_ * D_

BT = 8192
NT = B_ // BT

AP_ = 32
L_ = 16
NW_ = 32
SLAB_ = B_ // 16
ROWS_ = B_ // NW_

_INV_SQRT2 = 1.0 / math.sqrt(2.0)


def _gelu(x):
    return 0.5 * x * (1.0 + jax.lax.erf(x * _INV_SQRT2))


def _sc_front_body(act_hbm, tb_hbm, out_hbm, act_v, out_v, part_v, allp_v,
                   tb_v, shared):
    c = lax.axis_index("c")
    s = lax.axis_index("s")
    wid = s * 2 + c

    pltpu.sync_copy(act_hbm.at[pl.ds(s * (SLAB_ * AP_), SLAB_ * AP_)], act_v)
    pltpu.sync_copy(tb_hbm, tb_v)

    def p1(i, carry):
        mn0, mn1, mx0, mx1 = carry
        a0 = act_v[pl.ds(i * AP_, L_)]
        a1 = act_v[pl.ds(i * AP_ + L_, L_)]
        return (jnp.minimum(mn0, a0), jnp.minimum(mn1, a1),
                jnp.maximum(mx0, a0), jnp.maximum(mx1, a1))

    big = jnp.full((L_,), jnp.inf, jnp.float32)
    mn0, mn1, mx0, mx1 = lax.fori_loop(0, SLAB_, p1, (big, big, -big, -big),
                                       unroll=8)
    part_v[pl.ds(0, L_)] = mn0
    part_v[pl.ds(L_, L_)] = mn1
    part_v[pl.ds(2 * L_, L_)] = mx0
    part_v[pl.ds(3 * L_, L_)] = mx1
    pltpu.sync_copy(part_v, shared.at[pl.ds(s * (4 * L_), 4 * L_)])
    plsc.subcore_barrier()
    pltpu.sync_copy(shared, allp_v)
    for j in range(16):
        o = j * 4 * L_
        if j == 0:
            mn0 = allp_v[pl.ds(o, L_)]
            mn1 = allp_v[pl.ds(o + L_, L_)]
            mx0 = allp_v[pl.ds(o + 2 * L_, L_)]
            mx1 = allp_v[pl.ds(o + 3 * L_, L_)]
        else:
            mn0 = jnp.minimum(mn0, allp_v[pl.ds(o, L_)])
            mn1 = jnp.minimum(mn1, allp_v[pl.ds(o + L_, L_)])
            mx0 = jnp.maximum(mx0, allp_v[pl.ds(o + 2 * L_, L_)])
            mx1 = jnp.maximum(mx1, allp_v[pl.ds(o + 3 * L_, L_)])
    df0 = mx0 - mn0
    df1 = mx1 - mn1

    th0 = [mn0 + df0 * tb_v[pl.ds(k * L_, L_)] for k in range(1, NB_ + 1)]
    th1 = [mn1 + df1 * tb_v[pl.ds(k * L_, L_)] for k in range(1, NB_ + 1)]

    base = c * ROWS_

    def p2(r, _):
        o = (base + r) * AP_
        a0 = act_v[pl.ds(o, L_)]
        a1 = act_v[pl.ds(o + L_, L_)]
        c0 = jnp.zeros((L_,), jnp.float32)
        c1 = jnp.zeros((L_,), jnp.float32)
        for k in range(NB_):
            c0 = c0 + jnp.where(th0[k] < a0, 1.0, 0.0)
            c1 = c1 + jnp.where(th1[k] < a1, 1.0, 0.0)
        out_v[pl.ds(r * AP_, L_)] = jnp.minimum(c0, float(NB_ - 1))
        out_v[pl.ds(r * AP_ + L_, L_)] = jnp.minimum(c1, float(NB_ - 1))
        return 0

    lax.fori_loop(0, ROWS_, p2, 0, unroll=4)
    pltpu.sync_copy(out_v, out_hbm.at[pl.ds(wid * (ROWS_ * AP_), ROWS_ * AP_)])


@functools.partial(
    pl.kernel,
    mesh=plsc.VectorSubcoreMesh(core_axis_name="c", subcore_axis_name="s"),
    out_type=jax.ShapeDtypeStruct((B_ * AP_,), jnp.float32),
    scratch_types=[
        pltpu.VMEM((SLAB_ * AP_,), jnp.float32),
        pltpu.VMEM((ROWS_ * AP_,), jnp.float32),
        pltpu.VMEM((4 * L_,), jnp.float32),
        pltpu.VMEM((16 * 4 * L_,), jnp.float32),
        pltpu.VMEM(((NB_ + 4) * L_,), jnp.float32),
        pltpu.VMEM_SHARED((16 * 4 * L_,), jnp.float32),
    ],
)
def _sc_front(act_hbm, tb_hbm, out_hbm, act_v, out_v, part_v, allp_v, tb_v,
              shared):
    _sc_front_body(act_hbm, tb_hbm, out_hbm, act_v, out_v, part_v, allp_v,
                   tb_v, shared)


def _minmax_body(act_ref, mm_ref):
    t = pl.program_id(0)
    act = act_ref[...]
    mn = jnp.min(act, axis=0, keepdims=True)
    mx = jnp.max(act, axis=0, keepdims=True)
    cur = jnp.concatenate([mn, -mx], axis=0)

    @pl.when(t == 0)
    def _init():
        mm_ref[...] = cur

    @pl.when(t != 0)
    def _acc():
        mm_ref[...] = jnp.minimum(mm_ref[...], cur)


def _main_body(tlin_ref, mm_ref, act_ref, tab_ref, W1_ref, b1_ref, W2_ref,
               b2_ref, out_ref, E_ref, M_ref):
    t = pl.program_id(0)

    @pl.when(t == 0)
    def _prep():
        er = jax.lax.broadcasted_iota(jnp.int32, (A_, C_), 0)
        ec = jax.lax.broadcasted_iota(jnp.int32, (A_, C_), 1)
        E_ref[...] = jnp.where(ec // NB_ == er, 1.0, 0.0).astype(jnp.bfloat16)
        dr = jax.lax.broadcasted_iota(jnp.int32, (D_, AD_), 0)
        dc = jax.lax.broadcasted_iota(jnp.int32, (D_, AD_), 1)
        erep = jnp.where(dc % D_ == dr, 1.0, 0.0)
        mr = jax.lax.broadcasted_iota(jnp.int32, (C_, AD_), 0)
        mc = jax.lax.broadcasted_iota(jnp.int32, (C_, AD_), 1)
        mask = jnp.where(mr // NB_ == mc // D_, 1.0, 0.0)
        t520 = jnp.dot(tab_ref[...], erep,
                       preferred_element_type=jnp.float32) * mask
        M_ref[...] = jnp.dot(t520, W1_ref[...],
                             preferred_element_type=jnp.float32
                             ).astype(jnp.bfloat16)

    act = act_ref[...]
    mn = mm_ref[0:1, :]
    diff = (-mm_ref[1:2, :]) - mn
    cnt = jnp.zeros_like(act)
    for k in range(1, NB_ + 1):
        th = mn + diff * tlin_ref[0, k]
        cnt = cnt + jnp.where(th < act, 1.0, 0.0)
    binv = jnp.minimum(cnt, float(NB_ - 1)).astype(jnp.bfloat16)
    bin_e = jnp.dot(binv, E_ref[...], preferred_element_type=jnp.float32)
    cidx = jax.lax.broadcasted_iota(jnp.int32, (1, C_), 1)
    jmod = (cidx % NB_).astype(jnp.float32)
    onehot = jnp.where(bin_e == jmod, 1.0, 0.0).astype(jnp.bfloat16)
    hpre = jnp.dot(onehot, M_ref[...],
                   preferred_element_type=jnp.float32) + b1_ref[...]
    h = _gelu(hpre).astype(jnp.bfloat16)
    o = jnp.dot(h, W2_ref[...], preferred_element_type=jnp.float32)
    out_ref[...] = _gelu(o + b2_ref[...])


def kernel(actions, tables, W1, b1, W2, b2):
    tab520 = tables.reshape(C_, D_)
    tlin = jnp.linspace(0.0, 1.0, NB_ + 1, dtype=jnp.float32)
    tb = jnp.broadcast_to(
        jnp.pad(tlin, (0, 3))[:, None],
        (NB_ + 4, L_)).astype(jnp.float32).reshape(-1)
    act_pad = jnp.pad(actions, ((0, 0), (0, AP_ - A_))).reshape(-1)
    b1r = b1.reshape(1, H_)
    b2r = b2.reshape(1, OUT_)

    mm = pl.pallas_call(
        _minmax_body,
        grid=(NT,),
        in_specs=[pl.BlockSpec((BT, A_), lambda t: (t, 0))],
        out_specs=pl.BlockSpec((2, A_), lambda t: (0, 0)),
        out_shape=jax.ShapeDtypeStruct((2, A_), jnp.float32),
        compiler_params=pltpu.CompilerParams(
            dimension_semantics=("arbitrary",)),
    )(actions)

    tlin2 = tlin.reshape(1, NB_ + 1)
    out = pl.pallas_call(
        _main_body,
        grid=(NT,),
        in_specs=[
            pl.BlockSpec((1, NB_ + 1), lambda t: (0, 0)),
            pl.BlockSpec((2, A_), lambda t: (0, 0)),
            pl.BlockSpec((BT, A_), lambda t: (t, 0)),
            pl.BlockSpec((C_, D_), lambda t: (0, 0)),
            pl.BlockSpec((AD_, H_), lambda t: (0, 0)),
            pl.BlockSpec((1, H_), lambda t: (0, 0)),
            pl.BlockSpec((H_, OUT_), lambda t: (0, 0)),
            pl.BlockSpec((1, OUT_), lambda t: (0, 0)),
        ],
        out_specs=pl.BlockSpec((BT, OUT_), lambda t: (t, 0)),
        out_shape=jax.ShapeDtypeStruct((B_, OUT_), jnp.float32),
        scratch_shapes=[
            pltpu.VMEM((A_, C_), jnp.bfloat16),
            pltpu.VMEM((C_, H_), jnp.bfloat16),
        ],
        compiler_params=pltpu.CompilerParams(
            dimension_semantics=("arbitrary",)),
    )(tlin2, mm, actions, tab520, W1, b1r, W2, b2r)
    return out
